# baseline (speedup 1.0000x reference)
import jax
import jax.numpy as jnp
from jax import lax
from jax.experimental import pallas as pl
from jax.experimental.pallas import tpu as pltpu
from jax.experimental.pallas import tpu_sc as plsc

N = 2097152
G = 128
V = G * G * G
NC = 2
NS = 16
NW = NC * NS
PER_W = N // NW
C = 512
CR = 512
L = 16

_params = pltpu.CompilerParams(
    needs_layout_passes=False, use_tc_tiling_on_sc=False)


def _splat_i32(v):
    return jnp.full((L,), v, dtype=jnp.int32)


def _dim_index_frac(u):
    u = jnp.clip(u, 0.0, 1.0 - 1e-07)
    x = u * jnp.float32(G - 1)
    i0 = x.astype(jnp.int32)
    w1 = x - i0.astype(jnp.float32)
    return i0, w1


def _pack_body(theta_hbm, packed_hbm, in_v, out_v, sem):
    wid = lax.axis_index("s") * NC + lax.axis_index("c")
    iota = lax.iota(jnp.int32, L)
    rows_per_w = V // NW

    def chunk_body(t, _):
        base = wid * rows_per_w + t * CR
        pltpu.sync_copy(theta_hbm.at[pl.ds(base * 3, CR * 3)],
                        in_v.at[pl.ds(0, CR * 3)])

        def group(g, _):
            vv = iota + g * L
            v3 = vv * 3
            zeros = jnp.zeros((L,), jnp.float32)
            for c in range(6):
                val = plsc.load_gather(in_v, [v3 + c])
                plsc.store_scatter(out_v, [vv, _splat_i32(c)], val)
            plsc.store_scatter(out_v, [vv, _splat_i32(6)], zeros)
            plsc.store_scatter(out_v, [vv, _splat_i32(7)], zeros)
            return 0

        lax.fori_loop(0, CR // L, group, 0)
        pltpu.sync_copy(out_v, packed_hbm.at[pl.ds(base, CR)])
        return 0

    lax.fori_loop(0, rows_per_w // CR, chunk_body, 0)


def _interp_body(coords_hbm, packed_hbm, out_hbm,
                 coords_v, idx_v, frac_v, rows_v, out_v, sem):
    wid = lax.axis_index("s") * NC + lax.axis_index("c")
    iota = lax.iota(jnp.int32, L)
    iota3 = iota * 3

    def chunk_body(t, _):
        base = wid * PER_W + t * C
        pltpu.sync_copy(coords_hbm.at[pl.ds(base * 3, C * 3)], coords_v)

        def make_pass1(s):
            def pass1(g, _):
                gg = s * (128 // L) + g
                off = iota3 + gg * (3 * L)
                x = plsc.load_gather(coords_v, [off])
                y = plsc.load_gather(coords_v, [off + 1])
                z = plsc.load_gather(coords_v, [off + 2])
                i0, wx1 = _dim_index_frac(x)
                j0, wy1 = _dim_index_frac(y)
                k0, wz1 = _dim_index_frac(z)
                a = i0 * (G * G) + j0 * G + k0
                b = a + G
                c = a + (G * G)
                d = c + G
                sl = pl.ds(g * L, L)
                idx_v[0, s, sl] = a
                idx_v[1, s, sl] = c
                idx_v[2, s, sl] = b
                idx_v[3, s, sl] = d
                gsl = pl.ds(gg * L, L)
                frac_v[0, gsl] = wx1
                frac_v[1, gsl] = wy1
                frac_v[2, gsl] = wz1
                return 0
            return pass1

        for s in range(C // 128):
            lax.fori_loop(0, 128 // L, make_pass1(s), 0)

        cps = [
            pltpu.async_copy(packed_hbm.at[idx_v.at[n, s]],
                             rows_v.at[n, pl.ds(s * 128, 128)], sem)
            for n in range(4)
            for s in range(C // 128)
        ]
        for cp in cps:
            cp.wait()

        def pass2(g, _):
            gsl = pl.ds(g * L, L)
            pid = iota + g * L
            wx1 = frac_v[0, gsl]
            wy1 = frac_v[1, gsl]
            wz1 = frac_v[2, gsl]
            wx0 = 1.0 - wx1
            wy0 = 1.0 - wy1
            wz0 = 1.0 - wz1
            wxy = (wx0 * wy0, wx1 * wy0, wx0 * wy1, wx1 * wy1)
            w = tuple(wz0 * t for t in wxy) + tuple(wz1 * t for t in wxy)
            for ch in range(3):
                acc = w[0] * plsc.load_gather(
                    rows_v, [_splat_i32(0), pid, _splat_i32(ch)])
                for n in range(1, 8):
                    acc = acc + w[n] * plsc.load_gather(
                        rows_v,
                        [_splat_i32(n % 4), pid, _splat_i32(ch + 3 * (n // 4))])
                plsc.store_scatter(out_v, [pid, _splat_i32(ch)], acc)
            return 0

        lax.fori_loop(0, C // L, pass2, 0)
        pltpu.sync_copy(out_v, out_hbm.at[pl.ds(base, C)])
        return 0

    lax.fori_loop(0, PER_W // C, chunk_body, 0)


@jax.jit
def kernel(coords, theta):
    theta_flat = theta.reshape(V * 3)
    coords_flat = coords.reshape(N * 3)
    mesh = plsc.VectorSubcoreMesh(core_axis_name="c", subcore_axis_name="s")

    pack = pl.kernel(
        _pack_body,
        out_type=jax.ShapeDtypeStruct((V, 8), jnp.float32),
        mesh=mesh,
        scratch_types=[
            pltpu.VMEM((CR * 3 + 8,), jnp.float32),
            pltpu.VMEM((CR, 8), jnp.float32),
            pltpu.SemaphoreType.DMA,
        ],
        compiler_params=_params,
    )
    packed = pack(theta_flat)

    interp = pl.kernel(
        _interp_body,
        out_type=jax.ShapeDtypeStruct((N, 3), jnp.float32),
        mesh=mesh,
        scratch_types=[
            pltpu.VMEM((C * 3,), jnp.float32),
            pltpu.VMEM((4, C // 128, 128), jnp.int32),
            pltpu.VMEM((3, C), jnp.float32),
            pltpu.VMEM((4, C, 8), jnp.float32),
            pltpu.VMEM((C, 3), jnp.float32),
            pltpu.SemaphoreType.DMA,
        ],
        compiler_params=_params,
    )
    return interp(coords_flat, packed)
